# baseline (speedup 1.0000x reference)
import functools

import jax
import jax.numpy as jnp
from jax import lax
from jax.experimental import pallas as pl
from jax.experimental.pallas import tpu as pltpu
from jax.experimental.pallas import tpu_sc as plsc

_EPS = 1e-12

B, C, H, W = 4, 64, 16, 16
HW = H * W
K = 1024
N = B * HW

_NC, _NS, _L = 2, 16, 16
_NW = _NC * _NS
_TPW = N // _NW


def _argmin_body(x_ref, cb_ref, idx_ref):
    cb = cb_ref[...]
    cn2 = jnp.sum(cb * cb, axis=1, keepdims=True)
    cb_hi = cb.astype(jnp.bfloat16)
    cb_lo = (cb - cb_hi.astype(jnp.float32)).astype(jnp.bfloat16)
    dn = (((1,), (0,)), ((), ()))
    for b in range(B):
        xb = x_ref[b]
        sumsq = jnp.sum(xb * xb, axis=0, keepdims=True)
        inv = 1.0 / jnp.maximum(jnp.sqrt(sumsq), _EPS)
        xb_hi = xb.astype(jnp.bfloat16)
        xb_lo = (xb - xb_hi.astype(jnp.float32)).astype(jnp.bfloat16)
        dots = (lax.dot_general(cb_hi, xb_hi, dn, preferred_element_type=jnp.float32)
                + (lax.dot_general(cb_hi, xb_lo, dn, preferred_element_type=jnp.float32)
                   + lax.dot_general(cb_lo, xb_hi, dn, preferred_element_type=jnp.float32)))
        scores = cn2 - (2.0 * inv) * dots
        mn = jnp.min(scores, axis=0, keepdims=True)
        ids = lax.broadcasted_iota(jnp.int32, scores.shape, 0)
        idx = jnp.min(jnp.where(scores == mn, ids, jnp.int32(K)), axis=0)
        idx_ref[pl.ds(b * HW, HW)] = idx


def _nearest_code_indices(x3, codebook):
    return pl.pallas_call(
        _argmin_body,
        out_shape=jax.ShapeDtypeStruct((N,), jnp.int32),
    )(x3, codebook)


def _gather_body(cb_hbm, idx_hbm, out_hbm, idx_v, rows_v, cb_sh, sem):
    sid = lax.axis_index("s")
    wid = sid * _NC + lax.axis_index("c")
    base = wid * _TPW
    rows_per_sub = K // _NS
    pltpu.sync_copy(cb_hbm.at[pl.ds(sid * rows_per_sub, rows_per_sub)],
                    cb_sh.at[pl.ds(sid * rows_per_sub, rows_per_sub)])
    pltpu.sync_copy(idx_hbm.at[pl.ds(base, _TPW)], idx_v)
    plsc.subcore_barrier()
    pltpu.async_copy(cb_sh.at[idx_v], rows_v, sem).wait()
    pltpu.sync_copy(rows_v, out_hbm.at[pl.ds(base, _TPW)])


@functools.lru_cache(maxsize=1)
def _gather_sc():
    return pl.kernel(
        _gather_body,
        out_type=jax.ShapeDtypeStruct((N, C), jnp.float32),
        mesh=plsc.VectorSubcoreMesh(core_axis_name="c", subcore_axis_name="s",
                                    num_cores=_NC, num_subcores=_NS),
        scratch_types=[
            pltpu.VMEM((_TPW,), jnp.int32),
            pltpu.VMEM((_TPW, C), jnp.float32),
            pltpu.VMEM_SHARED((K, C), jnp.float32),
            pltpu.SemaphoreType.DMA,
        ],
        compiler_params=pltpu.CompilerParams(use_tc_tiling_on_sc=False,
                                             needs_layout_passes=False),
    )


def kernel(x, codebook):
    x3 = x.reshape(B, C, HW)
    idx = _nearest_code_indices(x3, codebook)
    q = _gather_sc()(codebook, idx)
    return jnp.transpose(q.reshape(B, H, W, C), (0, 3, 1, 2))


# speedup vs baseline: 1.0212x; 1.0212x over previous
import functools

import jax
import jax.numpy as jnp
from jax import lax
from jax.experimental import pallas as pl
from jax.experimental.pallas import tpu as pltpu
from jax.experimental.pallas import tpu_sc as plsc

_EPS = 1e-12

B, C, H, W = 4, 64, 16, 16
HW = H * W
K = 1024
N = B * HW

_NC, _NS, _L = 2, 16, 16
_NW = _NC * _NS
_TPW = N // _NW


def _argmin_body(z_ref, cb_ref, idx_ref):
    cb = cb_ref[...]
    z = z_ref[...]
    cn2 = jnp.sum(cb * cb, axis=1, keepdims=True)
    sumsq = jnp.sum(z * z, axis=1, keepdims=True)
    inv = 1.0 / jnp.maximum(jnp.sqrt(sumsq), _EPS)
    invr = lax.transpose(inv, (1, 0))
    cb_hi = cb.astype(jnp.bfloat16)
    cb_lo = (cb - cb_hi.astype(jnp.float32)).astype(jnp.bfloat16)
    z_hi = z.astype(jnp.bfloat16)
    z_lo = (z - z_hi.astype(jnp.float32)).astype(jnp.bfloat16)
    dn = (((1,), (1,)), ((), ()))
    dots = (lax.dot_general(cb_hi, z_hi, dn, preferred_element_type=jnp.float32)
            + (lax.dot_general(cb_hi, z_lo, dn, preferred_element_type=jnp.float32)
               + lax.dot_general(cb_lo, z_hi, dn, preferred_element_type=jnp.float32)))
    scores = cn2 - (2.0 * invr) * dots
    mn = jnp.min(scores, axis=0, keepdims=True)
    ids = lax.broadcasted_iota(jnp.int32, scores.shape, 0)
    idx = jnp.min(jnp.where(scores == mn, ids, jnp.int32(K)), axis=0)
    idx_ref[...] = idx


def _nearest_code_indices(z2d, codebook):
    return pl.pallas_call(
        _argmin_body,
        out_shape=jax.ShapeDtypeStruct((N,), jnp.int32),
    )(z2d, codebook)


def _gather_body(cb_hbm, idx_hbm, out_hbm, idx_v, rows_v, cb_sh, sem):
    sid = lax.axis_index("s")
    wid = sid * _NC + lax.axis_index("c")
    base = wid * _TPW
    rows_per_sub = K // _NS
    pltpu.sync_copy(cb_hbm.at[pl.ds(sid * rows_per_sub, rows_per_sub)],
                    cb_sh.at[pl.ds(sid * rows_per_sub, rows_per_sub)])
    pltpu.sync_copy(idx_hbm.at[pl.ds(base, _TPW)], idx_v)
    plsc.subcore_barrier()
    pltpu.async_copy(cb_sh.at[idx_v], rows_v, sem).wait()
    pltpu.sync_copy(rows_v, out_hbm.at[pl.ds(base, _TPW)])


@functools.lru_cache(maxsize=1)
def _gather_sc():
    return pl.kernel(
        _gather_body,
        out_type=jax.ShapeDtypeStruct((N, C), jnp.float32),
        mesh=plsc.VectorSubcoreMesh(core_axis_name="c", subcore_axis_name="s",
                                    num_cores=_NC, num_subcores=_NS),
        scratch_types=[
            pltpu.VMEM((_TPW,), jnp.int32),
            pltpu.VMEM((_TPW, C), jnp.float32),
            pltpu.VMEM_SHARED((K, C), jnp.float32),
            pltpu.SemaphoreType.DMA,
        ],
        compiler_params=pltpu.CompilerParams(use_tc_tiling_on_sc=False,
                                             needs_layout_passes=False),
    )


def kernel(x, codebook):
    z2d = jnp.transpose(x, (0, 2, 3, 1)).reshape(N, C)
    idx = _nearest_code_indices(z2d, codebook)
    q = _gather_sc()(codebook, idx)
    return jnp.transpose(q.reshape(B, H, W, C), (0, 3, 1, 2))
